# Initial kernel scaffold; baseline (speedup 1.0000x reference)
import jax
import jax.numpy as jnp
from jax.experimental import pallas as pl

_B, _C, _L, _E, _D = 64, 32, 2048, 8, 768
_TAU = 1.0


def _gelu(v):
    return 0.5 * v * (1.0 + jax.lax.erf(v * 0.7071067811865476))


def _stage1(x_ref, w_ref, b_ref, feat_ref, pooled_ref):
    X = x_ref[0]
    z = jnp.zeros((_C, 1), jnp.float32)
    Xl = jnp.concatenate([z, X[:, :-1]], axis=1)
    Xr = jnp.concatenate([X[:, 1:], z], axis=1)
    Xs = jnp.concatenate([Xl, X, Xr], axis=0)
    H = jax.lax.dot_general(w_ref[...], Xs, (((1,), (0,)), ((), ())),
                            preferred_element_type=jnp.float32)
    H = _gelu(H + b_ref[...])
    Hp = jnp.mean(H.reshape(64, 16, 128), axis=2)
    feat_ref[0] = Hp
    pooled_ref[...] = jnp.mean(X, axis=1)[None, :]


def _stage2(feat_ref, pooled_ref, bnw_ref, bnb_ref, bnm_ref, bnv_ref,
            fc1w_ref, fc1b_ref, fc2w_ref, fc2b_ref, gum_ref,
            expw_ref, expb_ref, out_ref):
    f = feat_ref[...]
    f = (f - bnm_ref[...]) * jax.lax.rsqrt(bnv_ref[...] + 1e-5) \
        * bnw_ref[...] + bnb_ref[...]
    h1 = _gelu(jnp.dot(f, fc1w_ref[...],
                       preferred_element_type=jnp.float32) + fc1b_ref[...])
    logits = jnp.dot(h1, fc2w_ref[...],
                     preferred_element_type=jnp.float32) + fc2b_ref[...]
    z = (logits + gum_ref[...]) / _TAU
    z = z - jnp.max(z, axis=1, keepdims=True)
    ez = jnp.exp(z)
    r = ez / jnp.sum(ez, axis=1, keepdims=True)

    col = jax.lax.broadcasted_iota(jnp.float32, (_B, _E), 1)
    m1 = jnp.max(r, axis=1, keepdims=True)
    i1 = jnp.min(jnp.where(r == m1, col, float(_E)), axis=1, keepdims=True)
    rm = jnp.where(col == i1, -jnp.inf, r)
    m2 = jnp.max(rm, axis=1, keepdims=True)
    i2 = jnp.min(jnp.where(rm == m2, col, float(_E)), axis=1, keepdims=True)
    s = m1 + m2 + 1e-8
    wfull = jnp.where(col == i1, m1 / s, 0.0) + jnp.where(col == i2, m2 / s, 0.0)

    ao = jnp.tanh(jnp.dot(pooled_ref[...], expw_ref[...],
                          preferred_element_type=jnp.float32)
                  + expb_ref[...])
    acc = jnp.zeros((_B, _D), jnp.float32)
    for e in range(_E):
        acc = acc + wfull[:, e:e + 1] * ao[:, e * _D:(e + 1) * _D]
    out_ref[...] = acc


def kernel(x, conv_w, conv_b, bn_w, bn_b, bn_mean, bn_var,
           fc1_w, fc1_b, fc2_w, fc2_b, gumbel, exp_w, exp_b):
    w96 = jnp.transpose(conv_w, (0, 2, 1)).reshape(64, 96)
    cb = conv_b.reshape(64, 1)
    expw2 = jnp.transpose(exp_w, (1, 0, 2)).reshape(_C, _E * _D)
    expb2 = exp_b.reshape(1, _E * _D)

    feat3, pooled = pl.pallas_call(
        _stage1,
        grid=(_B,),
        in_specs=[
            pl.BlockSpec((1, _C, _L), lambda i: (i, 0, 0)),
            pl.BlockSpec((64, 96), lambda i: (0, 0)),
            pl.BlockSpec((64, 1), lambda i: (0, 0)),
        ],
        out_specs=[
            pl.BlockSpec((1, 64, 16), lambda i: (i, 0, 0)),
            pl.BlockSpec((1, _C), lambda i: (i, 0)),
        ],
        out_shape=[
            jax.ShapeDtypeStruct((_B, 64, 16), jnp.float32),
            jax.ShapeDtypeStruct((_B, _C), jnp.float32),
        ],
    )(x, w96, cb)

    feats = feat3.reshape(_B, 64 * 16)

    out = pl.pallas_call(
        _stage2,
        out_shape=jax.ShapeDtypeStruct((_B, _D), jnp.float32),
    )(feats, pooled,
      bn_w.reshape(1, -1), bn_b.reshape(1, -1),
      bn_mean.reshape(1, -1), bn_var.reshape(1, -1),
      fc1_w, fc1_b.reshape(1, -1), fc2_w, fc2_b.reshape(1, -1),
      gumbel, expw2, expb2)
    return out


# speedup vs baseline: 3.3113x; 3.3113x over previous
import jax
import jax.numpy as jnp
from jax.experimental import pallas as pl

_B, _C, _L, _E, _D = 64, 32, 2048, 8, 768
_TAU = 1.0


def _gelu(v):
    return 0.5 * v * (1.0 + jax.lax.erf(v * 0.7071067811865476))


def _stage1(x_ref, w_ref, b_ref, feat_ref, pooled_ref):
    X = x_ref[0]
    z = jnp.zeros((_C, 1), jnp.float32)
    Xl = jnp.concatenate([z, X[:, :-1]], axis=1)
    Xr = jnp.concatenate([X[:, 1:], z], axis=1)
    Xs = jnp.concatenate([Xl, X, Xr], axis=0)
    H = jax.lax.dot_general(w_ref[...], Xs, (((1,), (0,)), ((), ())),
                            preferred_element_type=jnp.float32)
    H = _gelu(H + b_ref[...])
    Hp = jnp.mean(H.reshape(64, 16, 128), axis=2)
    feat_ref[0] = Hp
    pooled_ref[0] = jnp.mean(X, axis=1)[None, :]


def _stage2(feat_ref, pooled_ref, bnw_ref, bnb_ref, bnm_ref, bnv_ref,
            fc1w_ref, fc1b_ref, fc2w_ref, fc2b_ref, gum_ref,
            expw_ref, expb_ref, out_ref):
    f = feat_ref[...]
    f = (f - bnm_ref[...]) * jax.lax.rsqrt(bnv_ref[...] + 1e-5) \
        * bnw_ref[...] + bnb_ref[...]
    h1 = _gelu(jnp.dot(f, fc1w_ref[...],
                       preferred_element_type=jnp.float32) + fc1b_ref[...])
    logits = jnp.dot(h1, fc2w_ref[...],
                     preferred_element_type=jnp.float32) + fc2b_ref[...]
    z = (logits + gum_ref[...]) / _TAU
    z = z - jnp.max(z, axis=1, keepdims=True)
    ez = jnp.exp(z)
    r = ez / jnp.sum(ez, axis=1, keepdims=True)

    col = jax.lax.broadcasted_iota(jnp.int32, (_B, _E), 1)
    m1 = jnp.max(r, axis=1, keepdims=True)
    i1 = jnp.min(jnp.where(r == m1, col, _E), axis=1, keepdims=True)
    rm = jnp.where(col == i1, -jnp.inf, r)
    m2 = jnp.max(rm, axis=1, keepdims=True)
    i2 = jnp.min(jnp.where(rm == m2, col, _E), axis=1, keepdims=True)
    s = m1 + m2 + 1e-8
    wfull = jnp.where(col == i1, m1 / s, 0.0) + jnp.where(col == i2, m2 / s, 0.0)

    ao = jnp.tanh(jnp.dot(pooled_ref[...], expw_ref[...],
                          preferred_element_type=jnp.float32)
                  + expb_ref[...])
    acc = jnp.zeros((_B, _D), jnp.float32)
    for e in range(_E):
        acc = acc + wfull[:, e:e + 1] * ao[:, e * _D:(e + 1) * _D]
    out_ref[...] = acc


def kernel(x, conv_w, conv_b, bn_w, bn_b, bn_mean, bn_var,
           fc1_w, fc1_b, fc2_w, fc2_b, gumbel, exp_w, exp_b):
    w96 = jnp.transpose(conv_w, (0, 2, 1)).reshape(64, 96)
    cb = conv_b.reshape(64, 1)
    expw2 = jnp.transpose(exp_w, (1, 0, 2)).reshape(_C, _E * _D)
    expb2 = exp_b.reshape(1, _E * _D)

    feat3, pooled = pl.pallas_call(
        _stage1,
        grid=(_B,),
        in_specs=[
            pl.BlockSpec((1, _C, _L), lambda i: (i, 0, 0)),
            pl.BlockSpec((64, 96), lambda i: (0, 0)),
            pl.BlockSpec((64, 1), lambda i: (0, 0)),
        ],
        out_specs=[
            pl.BlockSpec((1, 64, 16), lambda i: (i, 0, 0)),
            pl.BlockSpec((1, 1, _C), lambda i: (i, 0, 0)),
        ],
        out_shape=[
            jax.ShapeDtypeStruct((_B, 64, 16), jnp.float32),
            jax.ShapeDtypeStruct((_B, 1, _C), jnp.float32),
        ],
    )(x, w96, cb)

    feats = feat3.reshape(_B, 64 * 16)
    pooled = pooled.reshape(_B, _C)

    out = pl.pallas_call(
        _stage2,
        out_shape=jax.ShapeDtypeStruct((_B, _D), jnp.float32),
    )(feats, pooled,
      bn_w.reshape(1, -1), bn_b.reshape(1, -1),
      bn_mean.reshape(1, -1), bn_var.reshape(1, -1),
      fc1_w, fc1_b.reshape(1, -1), fc2_w, fc2_b.reshape(1, -1),
      gumbel, expw2, expb2)
    return out


# speedup vs baseline: 5.7604x; 1.7396x over previous
import jax
import jax.numpy as jnp
from jax.experimental import pallas as pl

_B, _C, _L, _E, _D = 64, 32, 2048, 8, 768
_TAU = 1.0


def _gelu(v):
    return 0.5 * v * (1.0 + jax.lax.erf(v * 0.7071067811865476))


_BB = 4


def _stage1(x_ref, w_ref, b_ref, p_ref, feat_ref, pooled_ref):
    W = w_ref[...]
    b = b_ref[...]
    P = p_ref[...]
    for j in range(_BB):
        X = x_ref[j]
        z = jnp.zeros((_C, 1), jnp.float32)
        Xl = jnp.concatenate([z, X[:, :-1]], axis=1)
        Xr = jnp.concatenate([X[:, 1:], z], axis=1)
        Xs = jnp.concatenate([Xl, X, Xr], axis=0)
        H = jax.lax.dot_general(W, Xs, (((1,), (0,)), ((), ())),
                                preferred_element_type=jnp.float32)
        H = _gelu(H + b)
        feat_ref[0, j] = jnp.dot(H, P, preferred_element_type=jnp.float32)
    pooled_ref[0] = jnp.mean(x_ref[...], axis=2)


def _stage2(feat_ref, pooled_ref, bnw_ref, bnb_ref, bnm_ref, bnv_ref,
            fc1w_ref, fc1b_ref, fc2w_ref, fc2b_ref, gum_ref,
            expw_ref, expb_ref, out_ref):
    f = feat_ref[...]
    f = (f - bnm_ref[...]) * jax.lax.rsqrt(bnv_ref[...] + 1e-5) \
        * bnw_ref[...] + bnb_ref[...]
    h1 = _gelu(jnp.dot(f, fc1w_ref[...],
                       preferred_element_type=jnp.float32) + fc1b_ref[...])
    logits = jnp.dot(h1, fc2w_ref[...],
                     preferred_element_type=jnp.float32) + fc2b_ref[...]
    z = (logits + gum_ref[...]) / _TAU
    z = z - jnp.max(z, axis=1, keepdims=True)
    ez = jnp.exp(z)
    r = ez / jnp.sum(ez, axis=1, keepdims=True)

    col = jax.lax.broadcasted_iota(jnp.int32, (_B, _E), 1)
    m1 = jnp.max(r, axis=1, keepdims=True)
    i1 = jnp.min(jnp.where(r == m1, col, _E), axis=1, keepdims=True)
    rm = jnp.where(col == i1, -jnp.inf, r)
    m2 = jnp.max(rm, axis=1, keepdims=True)
    i2 = jnp.min(jnp.where(rm == m2, col, _E), axis=1, keepdims=True)
    s = m1 + m2 + 1e-8
    wfull = jnp.where(col == i1, m1 / s, 0.0) + jnp.where(col == i2, m2 / s, 0.0)

    ao = jnp.tanh(jnp.dot(pooled_ref[...], expw_ref[...],
                          preferred_element_type=jnp.float32)
                  + expb_ref[...])
    acc = jnp.zeros((_B, _D), jnp.float32)
    for e in range(_E):
        acc = acc + wfull[:, e:e + 1] * ao[:, e * _D:(e + 1) * _D]
    out_ref[...] = acc


def kernel(x, conv_w, conv_b, bn_w, bn_b, bn_mean, bn_var,
           fc1_w, fc1_b, fc2_w, fc2_b, gumbel, exp_w, exp_b):
    w96 = jnp.transpose(conv_w, (0, 2, 1)).reshape(64, 96)
    cb = conv_b.reshape(64, 1)
    expw2 = jnp.transpose(exp_w, (1, 0, 2)).reshape(_C, _E * _D)
    expb2 = exp_b.reshape(1, _E * _D)

    pool_mat = (jnp.arange(_L)[:, None] // 128
                == jnp.arange(16)[None, :]).astype(jnp.float32) / 128.0

    ng = _B // _BB
    feat4, pooled = pl.pallas_call(
        _stage1,
        grid=(ng,),
        in_specs=[
            pl.BlockSpec((_BB, _C, _L), lambda i: (i, 0, 0)),
            pl.BlockSpec((64, 96), lambda i: (0, 0)),
            pl.BlockSpec((64, 1), lambda i: (0, 0)),
            pl.BlockSpec((_L, 16), lambda i: (0, 0)),
        ],
        out_specs=[
            pl.BlockSpec((1, _BB, 64, 16), lambda i: (i, 0, 0, 0)),
            pl.BlockSpec((1, _BB, _C), lambda i: (i, 0, 0)),
        ],
        out_shape=[
            jax.ShapeDtypeStruct((ng, _BB, 64, 16), jnp.float32),
            jax.ShapeDtypeStruct((ng, _BB, _C), jnp.float32),
        ],
    )(x, w96, cb, pool_mat)

    feats = feat4.reshape(_B, 64 * 16)
    pooled = pooled.reshape(_B, _C)

    out = pl.pallas_call(
        _stage2,
        out_shape=jax.ShapeDtypeStruct((_B, _D), jnp.float32),
    )(feats, pooled,
      bn_w.reshape(1, -1), bn_b.reshape(1, -1),
      bn_mean.reshape(1, -1), bn_var.reshape(1, -1),
      fc1_w, fc1_b.reshape(1, -1), fc2_w, fc2_b.reshape(1, -1),
      gumbel, expw2, expb2)
    return out


# speedup vs baseline: 6.3544x; 1.1031x over previous
import jax
import jax.numpy as jnp
from jax.experimental import pallas as pl
from jax.experimental.pallas import tpu as pltpu

_B, _C, _L, _E, _D = 64, 32, 2048, 8, 768
_TAU = 1.0
_BB = 8
_NG = _B // _BB


def _gelu(v):
    return 0.5 * v * (1.0 + jax.lax.erf(v * 0.7071067811865476))


def _body(x_ref, w_ref, b_ref, p_ref, bnw_ref, bnb_ref, bnm_ref, bnv_ref,
          fc1w_ref, fc1b_ref, fc2w_ref, fc2b_ref, gum_ref, expw_ref,
          expb_ref, out_ref, feat_s, pooled_s):
    i = pl.program_id(0)
    W = w_ref[...]
    b = b_ref[...]
    P = p_ref[...]
    for j in range(_BB):
        X = x_ref[j]
        z = jnp.zeros((_C, 1), jnp.float32)
        Xl = jnp.concatenate([z, X[:, :-1]], axis=1)
        Xr = jnp.concatenate([X[:, 1:], z], axis=1)
        Xs = jnp.concatenate([Xl, X, Xr], axis=0)
        H = jax.lax.dot_general(W, Xs, (((1,), (0,)), ((), ())),
                                preferred_element_type=jnp.float32)
        H = _gelu(H + b)
        feat_s[i * _BB + j] = jnp.dot(H, P, preferred_element_type=jnp.float32)
    pooled_s[pl.ds(i * _BB, _BB)] = jnp.mean(x_ref[...], axis=2)

    @pl.when(i == _NG - 1)
    def _stage2():
        f = feat_s[...].reshape(_B, 64 * 16)
        f = (f - bnm_ref[...]) * jax.lax.rsqrt(bnv_ref[...] + 1e-5) \
            * bnw_ref[...] + bnb_ref[...]
        h1 = _gelu(jnp.dot(f, fc1w_ref[...],
                           preferred_element_type=jnp.float32)
                   + fc1b_ref[...])
        logits = jnp.dot(h1, fc2w_ref[...],
                         preferred_element_type=jnp.float32) + fc2b_ref[...]
        zz = (logits + gum_ref[...]) / _TAU
        zz = zz - jnp.max(zz, axis=1, keepdims=True)
        ez = jnp.exp(zz)
        r = ez / jnp.sum(ez, axis=1, keepdims=True)

        col = jax.lax.broadcasted_iota(jnp.int32, (_B, _E), 1)
        m1 = jnp.max(r, axis=1, keepdims=True)
        i1 = jnp.min(jnp.where(r == m1, col, _E), axis=1, keepdims=True)
        rm = jnp.where(col == i1, -jnp.inf, r)
        m2 = jnp.max(rm, axis=1, keepdims=True)
        i2 = jnp.min(jnp.where(rm == m2, col, _E), axis=1, keepdims=True)
        s = m1 + m2 + 1e-8
        wfull = (jnp.where(col == i1, m1 / s, 0.0)
                 + jnp.where(col == i2, m2 / s, 0.0))

        ao = jnp.tanh(jnp.dot(pooled_s[...], expw_ref[...],
                              preferred_element_type=jnp.float32)
                      + expb_ref[...])
        acc = jnp.zeros((_B, _D), jnp.float32)
        for e in range(_E):
            acc = acc + wfull[:, e:e + 1] * ao[:, e * _D:(e + 1) * _D]
        out_ref[...] = acc


def kernel(x, conv_w, conv_b, bn_w, bn_b, bn_mean, bn_var,
           fc1_w, fc1_b, fc2_w, fc2_b, gumbel, exp_w, exp_b):
    w96 = jnp.transpose(conv_w, (0, 2, 1)).reshape(64, 96)
    cb = conv_b.reshape(64, 1)
    expw2 = jnp.transpose(exp_w, (1, 0, 2)).reshape(_C, _E * _D)
    expb2 = exp_b.reshape(1, _E * _D)
    pool_mat = (jnp.arange(_L)[:, None] // 128
                == jnp.arange(16)[None, :]).astype(jnp.float32) / 128.0

    cst = lambda *dims: pl.BlockSpec(dims, lambda i: (0,) * len(dims))
    out = pl.pallas_call(
        _body,
        grid=(_NG,),
        in_specs=[
            pl.BlockSpec((_BB, _C, _L), lambda i: (i, 0, 0)),
            cst(64, 96), cst(64, 1), cst(_L, 16),
            cst(1, 1024), cst(1, 1024), cst(1, 1024), cst(1, 1024),
            cst(1024, 128), cst(1, 128), cst(128, _E), cst(1, _E),
            cst(_B, _E), cst(_C, _E * _D), cst(1, _E * _D),
        ],
        out_specs=pl.BlockSpec((_B, _D), lambda i: (0, 0)),
        out_shape=jax.ShapeDtypeStruct((_B, _D), jnp.float32),
        scratch_shapes=[
            pltpu.VMEM((_B, 64, 16), jnp.float32),
            pltpu.VMEM((_B, _C), jnp.float32),
        ],
    )(x, w96, cb, pool_mat,
      bn_w.reshape(1, -1), bn_b.reshape(1, -1),
      bn_mean.reshape(1, -1), bn_var.reshape(1, -1),
      fc1_w, fc1_b.reshape(1, -1), fc2_w, fc2_b.reshape(1, -1),
      gumbel, expw2, expb2)
    return out


# speedup vs baseline: 6.9291x; 1.0904x over previous
import jax
import jax.numpy as jnp
from jax.experimental import pallas as pl
from jax.experimental.pallas import tpu as pltpu

_B, _C, _L, _E, _D = 64, 32, 2048, 8, 768
_TAU = 1.0
_BB = 8
_NG = _B // _BB


def _gelu(v):
    return 0.5 * v * (1.0 + jax.lax.erf(v * 0.7071067811865476))


def _body(x_ref, w_ref, b_ref, p_ref, bnw_ref, bnb_ref, bnm_ref, bnv_ref,
          fc1w_ref, fc1b_ref, fc2w_ref, fc2b_ref, gum_ref, expw_ref,
          expb_ref, out_ref, feat_s, pooled_s, xs_s):
    i = pl.program_id(0)
    W = w_ref[...]
    b = b_ref[...]
    P = p_ref[...]
    for j in range(_BB):
        X = x_ref[j]
        z = jnp.zeros((_C, 1), jnp.float32)
        Xl = jnp.concatenate([z, X[:, :-1]], axis=1)
        Xr = jnp.concatenate([X[:, 1:], z], axis=1)
        xs_s[:, pl.ds(j * _L, _L)] = jnp.concatenate([Xl, X, Xr], axis=0)
    H = jax.lax.dot_general(W, xs_s[...], (((1,), (0,)), ((), ())),
                            preferred_element_type=jnp.float32)
    H = _gelu(H + b)
    for j in range(_BB):
        feat_s[i * _BB + j] = jnp.dot(H[:, j * _L:(j + 1) * _L], P,
                                      preferred_element_type=jnp.float32)
    pooled_s[pl.ds(i * _BB, _BB)] = jnp.mean(x_ref[...], axis=2)

    @pl.when(i == _NG - 1)
    def _stage2():
        f = feat_s[...].reshape(_B, 64 * 16)
        f = (f - bnm_ref[...]) * jax.lax.rsqrt(bnv_ref[...] + 1e-5) \
            * bnw_ref[...] + bnb_ref[...]
        h1 = _gelu(jnp.dot(f, fc1w_ref[...],
                           preferred_element_type=jnp.float32)
                   + fc1b_ref[...])
        logits = jnp.dot(h1, fc2w_ref[...],
                         preferred_element_type=jnp.float32) + fc2b_ref[...]
        zz = (logits + gum_ref[...]) / _TAU
        zz = zz - jnp.max(zz, axis=1, keepdims=True)
        ez = jnp.exp(zz)
        r = ez / jnp.sum(ez, axis=1, keepdims=True)

        col = jax.lax.broadcasted_iota(jnp.int32, (_B, _E), 1)
        m1 = jnp.max(r, axis=1, keepdims=True)
        i1 = jnp.min(jnp.where(r == m1, col, _E), axis=1, keepdims=True)
        rm = jnp.where(col == i1, -jnp.inf, r)
        m2 = jnp.max(rm, axis=1, keepdims=True)
        i2 = jnp.min(jnp.where(rm == m2, col, _E), axis=1, keepdims=True)
        s = m1 + m2 + 1e-8
        wfull = (jnp.where(col == i1, m1 / s, 0.0)
                 + jnp.where(col == i2, m2 / s, 0.0))

        ao = jnp.tanh(jnp.dot(pooled_s[...], expw_ref[...],
                              preferred_element_type=jnp.float32)
                      + expb_ref[...])
        acc = jnp.zeros((_B, _D), jnp.float32)
        for e in range(_E):
            acc = acc + wfull[:, e:e + 1] * ao[:, e * _D:(e + 1) * _D]
        out_ref[...] = acc


def kernel(x, conv_w, conv_b, bn_w, bn_b, bn_mean, bn_var,
           fc1_w, fc1_b, fc2_w, fc2_b, gumbel, exp_w, exp_b):
    w96 = jnp.transpose(conv_w, (0, 2, 1)).reshape(64, 96)
    cb = conv_b.reshape(64, 1)
    expw2 = jnp.transpose(exp_w, (1, 0, 2)).reshape(_C, _E * _D)
    expb2 = exp_b.reshape(1, _E * _D)
    pool_mat = (jnp.arange(_L)[:, None] // 128
                == jnp.arange(16)[None, :]).astype(jnp.float32) / 128.0

    cst = lambda *dims: pl.BlockSpec(dims, lambda i: (0,) * len(dims))
    out = pl.pallas_call(
        _body,
        grid=(_NG,),
        in_specs=[
            pl.BlockSpec((_BB, _C, _L), lambda i: (i, 0, 0)),
            cst(64, 96), cst(64, 1), cst(_L, 16),
            cst(1, 1024), cst(1, 1024), cst(1, 1024), cst(1, 1024),
            cst(1024, 128), cst(1, 128), cst(128, _E), cst(1, _E),
            cst(_B, _E), cst(_C, _E * _D), cst(1, _E * _D),
        ],
        out_specs=pl.BlockSpec((_B, _D), lambda i: (0, 0)),
        out_shape=jax.ShapeDtypeStruct((_B, _D), jnp.float32),
        scratch_shapes=[
            pltpu.VMEM((_B, 64, 16), jnp.float32),
            pltpu.VMEM((_B, _C), jnp.float32),
            pltpu.VMEM((3 * _C, _BB * _L), jnp.float32),
        ],
    )(x, w96, cb, pool_mat,
      bn_w.reshape(1, -1), bn_b.reshape(1, -1),
      bn_mean.reshape(1, -1), bn_var.reshape(1, -1),
      fc1_w, fc1_b.reshape(1, -1), fc2_w, fc2_b.reshape(1, -1),
      gumbel, expw2, expb2)
    return out


# speedup vs baseline: 7.0630x; 1.0193x over previous
import jax
import jax.numpy as jnp
from jax.experimental import pallas as pl
from jax.experimental.pallas import tpu as pltpu

_B, _C, _L, _E, _D = 64, 32, 2048, 8, 768
_TAU = 1.0
_BB = 16
_NG = _B // _BB


def _gelu(v):
    return 0.5 * v * (1.0 + jax.lax.erf(v * 0.7071067811865476))


def _body(x_ref, w_ref, b_ref, p_ref, bnw_ref, bnb_ref, bnm_ref, bnv_ref,
          fc1w_ref, fc1b_ref, fc2w_ref, fc2b_ref, gum_ref, expw_ref,
          expb_ref, out_ref, feat_s, pooled_s, xs_s):
    i = pl.program_id(0)
    W = w_ref[...]
    b = b_ref[...]
    P = p_ref[...]
    for j in range(_BB):
        X = x_ref[j]
        z = jnp.zeros((_C, 1), jnp.float32)
        Xl = jnp.concatenate([z, X[:, :-1]], axis=1)
        Xr = jnp.concatenate([X[:, 1:], z], axis=1)
        xs_s[:, pl.ds(j * _L, _L)] = jnp.concatenate([Xl, X, Xr], axis=0)
    H = jax.lax.dot_general(W, xs_s[...], (((1,), (0,)), ((), ())),
                            preferred_element_type=jnp.float32)
    H = _gelu(H + b)
    for j in range(_BB):
        feat_s[i * _BB + j] = jnp.dot(H[:, j * _L:(j + 1) * _L], P,
                                      preferred_element_type=jnp.float32)
    pooled_s[pl.ds(i * _BB, _BB)] = jnp.mean(x_ref[...], axis=2)

    @pl.when(i == _NG - 1)
    def _stage2():
        f = feat_s[...].reshape(_B, 64 * 16)
        f = (f - bnm_ref[...]) * jax.lax.rsqrt(bnv_ref[...] + 1e-5) \
            * bnw_ref[...] + bnb_ref[...]
        h1 = _gelu(jnp.dot(f, fc1w_ref[...],
                           preferred_element_type=jnp.float32)
                   + fc1b_ref[...])
        logits = jnp.dot(h1, fc2w_ref[...],
                         preferred_element_type=jnp.float32) + fc2b_ref[...]
        zz = (logits + gum_ref[...]) / _TAU
        zz = zz - jnp.max(zz, axis=1, keepdims=True)
        ez = jnp.exp(zz)
        r = ez / jnp.sum(ez, axis=1, keepdims=True)

        col = jax.lax.broadcasted_iota(jnp.int32, (_B, _E), 1)
        m1 = jnp.max(r, axis=1, keepdims=True)
        i1 = jnp.min(jnp.where(r == m1, col, _E), axis=1, keepdims=True)
        rm = jnp.where(col == i1, -jnp.inf, r)
        m2 = jnp.max(rm, axis=1, keepdims=True)
        i2 = jnp.min(jnp.where(rm == m2, col, _E), axis=1, keepdims=True)
        s = m1 + m2 + 1e-8
        wfull = (jnp.where(col == i1, m1 / s, 0.0)
                 + jnp.where(col == i2, m2 / s, 0.0))

        ao = jnp.tanh(jnp.dot(pooled_s[...], expw_ref[...],
                              preferred_element_type=jnp.float32)
                      + expb_ref[...])
        acc = jnp.zeros((_B, _D), jnp.float32)
        for e in range(_E):
            acc = acc + wfull[:, e:e + 1] * ao[:, e * _D:(e + 1) * _D]
        out_ref[...] = acc


def kernel(x, conv_w, conv_b, bn_w, bn_b, bn_mean, bn_var,
           fc1_w, fc1_b, fc2_w, fc2_b, gumbel, exp_w, exp_b):
    w96 = jnp.transpose(conv_w, (0, 2, 1)).reshape(64, 96)
    cb = conv_b.reshape(64, 1)
    expw2 = jnp.transpose(exp_w, (1, 0, 2)).reshape(_C, _E * _D)
    expb2 = exp_b.reshape(1, _E * _D)
    pool_mat = (jnp.arange(_L)[:, None] // 128
                == jnp.arange(16)[None, :]).astype(jnp.float32) / 128.0

    cst = lambda *dims: pl.BlockSpec(dims, lambda i: (0,) * len(dims))
    out = pl.pallas_call(
        _body,
        grid=(_NG,),
        in_specs=[
            pl.BlockSpec((_BB, _C, _L), lambda i: (i, 0, 0)),
            cst(64, 96), cst(64, 1), cst(_L, 16),
            cst(1, 1024), cst(1, 1024), cst(1, 1024), cst(1, 1024),
            cst(1024, 128), cst(1, 128), cst(128, _E), cst(1, _E),
            cst(_B, _E), cst(_C, _E * _D), cst(1, _E * _D),
        ],
        out_specs=pl.BlockSpec((_B, _D), lambda i: (0, 0)),
        out_shape=jax.ShapeDtypeStruct((_B, _D), jnp.float32),
        scratch_shapes=[
            pltpu.VMEM((_B, 64, 16), jnp.float32),
            pltpu.VMEM((_B, _C), jnp.float32),
            pltpu.VMEM((3 * _C, _BB * _L), jnp.float32),
        ],
    )(x, w96, cb, pool_mat,
      bn_w.reshape(1, -1), bn_b.reshape(1, -1),
      bn_mean.reshape(1, -1), bn_var.reshape(1, -1),
      fc1_w, fc1_b.reshape(1, -1), fc2_w, fc2_b.reshape(1, -1),
      gumbel, expw2, expb2)
    return out
